# baseline (speedup 1.0000x reference)
import functools

import jax
import jax.numpy as jnp
from jax import lax
from jax.experimental import pallas as pl
from jax.experimental.pallas import tpu as pltpu
from jax.experimental.pallas import tpu_sc as plsc

NC = 2
NS = 16
NW = NC * NS
CHUNK = 128


@functools.lru_cache(maxsize=None)
def _make_gather(batch, du, dm):
    bpw = batch // NW
    nchunks = bpw // CHUNK
    mesh = plsc.VectorSubcoreMesh(core_axis_name="c", subcore_axis_name="s")

    @functools.partial(
        pl.kernel,
        mesh=mesh,
        compiler_params=pltpu.CompilerParams(use_tc_tiling_on_sc=False),
        out_type=[
            jax.ShapeDtypeStruct((batch, du), jnp.float32),
            jax.ShapeDtypeStruct((batch, dm), jnp.float32),
        ],
        scratch_types=[
            pltpu.VMEM((nchunks, CHUNK), jnp.int32),
            pltpu.VMEM((nchunks, CHUNK), jnp.int32),
            pltpu.VMEM((bpw, du), jnp.float32),
            pltpu.VMEM((bpw, dm), jnp.float32),
            pltpu.SemaphoreType.DMA,
            pltpu.SemaphoreType.DMA,
        ],
    )
    def gather_k(uid_hbm, mid_hbm, utab_hbm, mtab_hbm, uout_hbm, mout_hbm,
                 uidx_v, midx_v, urows_v, mrows_v, usem, msem):
        wid = lax.axis_index("s") * NC + lax.axis_index("c")
        base = wid * bpw
        pltpu.sync_copy(uid_hbm.at[wid], uidx_v)
        pltpu.sync_copy(mid_hbm.at[wid], midx_v)
        copies = []
        for c in range(nchunks):
            copies.append(pltpu.async_copy(
                utab_hbm.at[uidx_v.at[c]],
                urows_v.at[pl.ds(c * CHUNK, CHUNK)], usem))
            copies.append(pltpu.async_copy(
                mtab_hbm.at[midx_v.at[c]],
                mrows_v.at[pl.ds(c * CHUNK, CHUNK)], msem))
        for cp in copies:
            cp.wait()
        pltpu.sync_copy(urows_v, uout_hbm.at[pl.ds(base, bpw)])
        pltpu.sync_copy(mrows_v, mout_hbm.at[pl.ds(base, bpw)])

    return gather_k


def _head_body(u_ref, m_ref, g_ref, gw_ref, gb_ref, fcw_ref, fcb_ref, o_ref):
    d = u_ref.shape[1]
    gd = g_ref.shape[1]
    genre_emb = jnp.dot(g_ref[...], gw_ref[...].T,
                        preferred_element_type=jnp.float32) + gb_ref[...]
    fcw = fcw_ref[...]
    wu = fcw[:, 0:d].T
    wm = fcw[:, d:2 * d].T
    wg = fcw[:, 2 * d:2 * d + d].T
    o_ref[...] = (
        jnp.dot(u_ref[...], wu, preferred_element_type=jnp.float32)
        + jnp.dot(m_ref[...], wm, preferred_element_type=jnp.float32)
        + jnp.dot(genre_emb, wg, preferred_element_type=jnp.float32)
        + fcb_ref[...]
    )


def kernel(user_id, movie_id, genre_features, user_table, movie_table,
           genre_W, genre_b, fc_W, fc_b):
    batch = user_id.shape[0]
    du = user_table.shape[1]
    dm = movie_table.shape[1]
    bpw = batch // NW
    nchunks = bpw // CHUNK

    uid = user_id.astype(jnp.int32).reshape(NW, nchunks, CHUNK)
    mid = movie_id.astype(jnp.int32).reshape(NW, nchunks, CHUNK)

    user_emb, movie_emb = _make_gather(batch, du, dm)(
        uid, mid, user_table, movie_table)

    head = pl.pallas_call(
        _head_body,
        out_shape=jax.ShapeDtypeStruct((batch, 1), jnp.float32),
    )
    return head(user_emb, movie_emb, genre_features,
                genre_W, genre_b.reshape(1, -1), fc_W, fc_b.reshape(1, 1))
